# baseline (speedup 1.0000x reference)
import functools

import jax
import jax.numpy as jnp
from jax import lax
from jax.experimental import pallas as pl
from jax.experimental.pallas import tpu as pltpu
from jax.experimental.pallas import tpu_sc as plsc

N = 10000
N_PAD = 10240
E_RAW = 320000
E_TOT = E_RAW + N
E_PAD = 331776
S_BIG = 384
BLK = 1024
GRID = N_PAD // BLK
STRIPE = N_PAD // 16


def _sc_agg_kernel(Dh, S, split_edges, off0, offc, do_den, n_chunks,
                   h_hbm, src_hbm, dst_hbm, as_hbm, ad_hbm, m_hbm,
                   zn_hbm, zd_hbm,
                   num_hbm, den_hbm,
                   acc, dacc, srcv, srcav, dstv, rowsv, wv, asg, adg, mv,
                   g0, g1, g2, a0, a1, a2, b0, b1, b2,
                   s0, s1, s2, d0, d1, d2):
    c = lax.axis_index("c")
    s = lax.axis_index("s")
    gsem = [g0, g1, g2]
    asem = [a0, a1, a2]
    bsem = [b0, b1, b2]
    ssem = [s0, s1, s2]
    dsem = [d0, d1, d2]
    subs = [(o, min(128, S - o)) for o in range(0, S, 128)]
    adjust = (off0 != 0) or (offc != 0)

    pltpu.sync_copy(zn_hbm, acc.at[pl.ds(s * STRIPE, STRIPE)])
    if do_den:
        pltpu.sync_copy(zd_hbm, dacc.at[pl.ds(s * STRIPE, STRIPE)])
    pltpu.sync_copy(m_hbm, mv)

    plsc.subcore_barrier()

    if split_edges:
        per_tile = E_PAD // 32
        base0 = (c * 16 + s) * per_tile
    else:
        per_tile = E_PAD // 16
        base0 = s * per_tile

    def gidx(b, d):
        ref = srcav if adjust else srcv
        o, l = subs[d]
        return ref.at[b, pl.ds(o, l)]

    def stage_a(i, b):
        base = base0 + i * S
        pltpu.sync_copy(src_hbm.at[pl.ds(base, S)], srcv.at[b])
        for d, (o, l) in enumerate(subs):
            pltpu.sync_copy(dst_hbm.at[pl.ds(base + o, l)], dstv.at[b, d])
        if adjust:
            for j in range(S // 16):
                sl = pl.ds(j * 16, 16)
                srcav[b, sl] = srcv[b, sl] + (off0 + c * offc)
        for d, (o, l) in enumerate(subs):
            pltpu.async_copy(h_hbm.at[gidx(b, d)], rowsv.at[b, pl.ds(o, l)], gsem[b])
            pltpu.async_copy(as_hbm.at[srcv.at[b, pl.ds(o, l)]], asg.at[b, pl.ds(o, l)], asem[b])
            pltpu.async_copy(ad_hbm.at[dstv.at[b, d]], adg.at[b, pl.ds(o, l)], bsem[b])

    def stage_b(i, b):
        base = base0 + i * S
        pltpu.make_async_copy(as_hbm.at[pl.ds(0, S)], asg.at[b], asem[b]).wait()
        pltpu.make_async_copy(ad_hbm.at[pl.ds(0, S)], adg.at[b], bsem[b]).wait()
        m16 = mv[...]
        for j in range(S // 16):
            sl = pl.ds(j * 16, 16)
            e = asg[b, sl] + adg[b, sl]
            e = jnp.where(e > 0, e, 0.2 * e)
            w = jnp.exp(e - m16)
            eid = base + j * 16 + lax.iota(jnp.int32, 16)
            wv[b, sl] = jnp.where(eid < E_TOT, w, 0.0)
        pltpu.make_async_copy(h_hbm.at[pl.ds(0, S)], rowsv.at[b], gsem[b]).wait()

        def scale_group(g, carry):
            w16 = wv[b, pl.ds(g * 16, 16)]
            for t in range(16):
                ws = w16[t]
                r = g * 16 + t
                for k in range(Dh // 16):
                    cl = pl.ds(k * 16, 16)
                    rowsv[b, r, cl] = rowsv[b, r, cl] * ws
            return carry

        lax.fori_loop(0, S // 16, scale_group, 0)
        for d, (o, l) in enumerate(subs):
            pltpu.async_copy(rowsv.at[b, pl.ds(o, l)], acc.at[dstv.at[b, d]],
                             ssem[b], add=True)
            if do_den:
                pltpu.async_copy(wv.at[b, pl.ds(o, l)], dacc.at[dstv.at[b, d]],
                                 dsem[b], add=True)

    def wait_scatters(b):
        pltpu.make_async_copy(h_hbm.at[pl.ds(0, S)], rowsv.at[b], ssem[b]).wait()
        if do_den:
            pltpu.make_async_copy(as_hbm.at[pl.ds(0, S)], wv.at[b], dsem[b]).wait()

    stage_a(0, 0)

    def outer(k, carry):
        for b in range(3):
            i = k * 3 + b
            bn = (b + 1) % 3
            if b == 2:
                wait_scatters(bn)
            else:
                @pl.when(k > 0)
                def _():
                    wait_scatters(bn)
            stage_a(i + 1, bn)
            stage_b(i, b)
        return carry

    lax.fori_loop(0, n_chunks // 3, outer, 0)

    wait_scatters((n_chunks - 2) % 3)
    wait_scatters((n_chunks - 1) % 3)
    bx = n_chunks % 3
    pltpu.make_async_copy(h_hbm.at[pl.ds(0, S)], rowsv.at[bx], gsem[bx]).wait()
    pltpu.make_async_copy(as_hbm.at[pl.ds(0, S)], asg.at[bx], asem[bx]).wait()
    pltpu.make_async_copy(ad_hbm.at[pl.ds(0, S)], adg.at[bx], bsem[bx]).wait()

    plsc.subcore_barrier()

    rs = pl.ds(s * STRIPE, STRIPE)
    pltpu.sync_copy(acc.at[rs], num_hbm.at[c, rs])
    if do_den:
        pltpu.sync_copy(dacc.at[rs], den_hbm.at[c, rs])


@functools.lru_cache(maxsize=None)
def _make_sc_agg(Dh, S, split_edges, off0, offc, do_den):
    n_chunks = (E_PAD // 32 if split_edges else E_PAD // 16) // S
    nd = (S + 127) // 128
    mesh = plsc.VectorSubcoreMesh(core_axis_name="c", subcore_axis_name="s")
    return pl.kernel(
        functools.partial(_sc_agg_kernel, Dh, S, split_edges, off0, offc,
                          do_den, n_chunks),
        mesh=mesh,
        out_type=[
            jax.ShapeDtypeStruct((2, N_PAD, Dh), jnp.float32),
            jax.ShapeDtypeStruct((2, N_PAD), jnp.float32),
        ],
        scratch_types=[
            pltpu.VMEM_SHARED((N_PAD, Dh), jnp.float32),
            pltpu.VMEM_SHARED((N_PAD,), jnp.float32),
            pltpu.VMEM((3, S), jnp.int32),
            pltpu.VMEM((3, S), jnp.int32),
            pltpu.VMEM((3, nd, 128), jnp.int32),
            pltpu.VMEM((3, S, Dh), jnp.float32),
            pltpu.VMEM((3, S), jnp.float32),
            pltpu.VMEM((3, S), jnp.float32),
            pltpu.VMEM((3, S), jnp.float32),
            pltpu.VMEM((16,), jnp.float32),
        ] + [pltpu.SemaphoreType.DMA] * 15,
        compiler_params=pltpu.CompilerParams(
            needs_layout_passes=False, use_tc_tiling_on_sc=False),
    )


def _sc_agg(h_table, srcp, dstp, as_t, ad_t, m16, Dh, S, split_edges,
            off0, offc, do_den=True):
    zn = jnp.zeros((STRIPE, Dh), jnp.float32)
    zd = jnp.zeros((STRIPE,), jnp.float32)
    fn = _make_sc_agg(Dh, S, split_edges, off0, offc, do_den)
    return fn(h_table, srcp, dstp, as_t, ad_t, m16, zn, zd)


def _leaky(t):
    return jnp.where(t > 0, t, 0.2 * t)


def _alphas_and_max(h, asr, adr, i, as_ref, ad_ref, m_ref, mx_ref):
    a_s = jnp.sum(h * asr, axis=1)
    a_d = jnp.sum(h * adr, axis=1)
    as_ref[...] = a_s[:, None]
    ad_ref[...] = a_d[:, None]
    bs = jnp.max(a_s)
    bd = jnp.max(a_d)

    @pl.when(i == 0)
    def _():
        mx_ref[0] = bs
        mx_ref[1] = bd

    @pl.when(i > 0)
    def _():
        mx_ref[0] = jnp.maximum(mx_ref[0], bs)
        mx_ref[1] = jnp.maximum(mx_ref[1], bd)

    m = _leaky(mx_ref[0] + mx_ref[1])
    m_ref[...] = jnp.full((1, 16), m, jnp.float32)


def _tc1_body(x_ref, w_ref, asr_ref, adr_ref,
              h_ref, as_ref, ad_ref, m_ref, mx_ref):
    i = pl.program_id(0)
    h = jnp.dot(x_ref[...], w_ref[...], preferred_element_type=jnp.float32)
    for q in range(4):
        h_ref[q] = h[:, 64 * q:64 * (q + 1)]
    _alphas_and_max(h, asr_ref[...], adr_ref[...], i, as_ref, ad_ref, m_ref, mx_ref)


@jax.jit
def _tc1(x_p, W1, asr, adr):
    return pl.pallas_call(
        _tc1_body,
        grid=(GRID,),
        in_specs=[
            pl.BlockSpec((BLK, 128), lambda i: (i, 0)),
            pl.BlockSpec((128, 256), lambda i: (0, 0)),
            pl.BlockSpec((1, 256), lambda i: (0, 0)),
            pl.BlockSpec((1, 256), lambda i: (0, 0)),
        ],
        out_specs=[
            pl.BlockSpec((4, BLK, 64), lambda i: (0, i, 0)),
            pl.BlockSpec((BLK, 1), lambda i: (i, 0)),
            pl.BlockSpec((BLK, 1), lambda i: (i, 0)),
            pl.BlockSpec((1, 16), lambda i: (0, 0)),
        ],
        out_shape=[
            jax.ShapeDtypeStruct((4, N_PAD, 64), jnp.float32),
            jax.ShapeDtypeStruct((N_PAD, 1), jnp.float32),
            jax.ShapeDtypeStruct((N_PAD, 1), jnp.float32),
            jax.ShapeDtypeStruct((1, 16), jnp.float32),
        ],
        scratch_shapes=[pltpu.SMEM((2,), jnp.float32)],
    )(x_p, W1, asr, adr)


def _mid_body(Dp, Dn, sum_parts, n0_ref, n1_ref, n2_ref, n3_ref,
              d0_ref, d1_ref, b_ref, w_ref,
              asr_ref, adr_ref, h_ref, as_ref, ad_ref, m_ref, mx_ref):
    i = pl.program_id(0)
    if sum_parts:
        num = n0_ref[0] + n1_ref[0]
        den = d0_ref[0] + d1_ref[0]
    else:
        num = jnp.concatenate(
            [n0_ref[0], n1_ref[0], n2_ref[0], n3_ref[0]], axis=1)
        den = d0_ref[0]
    x = num / den + b_ref[...]
    x = jnp.maximum(x, 0.0)
    row = i * BLK + lax.broadcasted_iota(jnp.int32, (BLK, 1), 0)
    x = jnp.where(row < N, x, 0.0)
    h = jnp.dot(x, w_ref[...], preferred_element_type=jnp.float32)
    if Dn < 16:
        h_ref[...] = jnp.concatenate(
            [h, jnp.zeros((BLK, 16 - Dn), jnp.float32)], axis=1)
    else:
        h_ref[...] = h
    _alphas_and_max(h, asr_ref[...], adr_ref[...], i, as_ref, ad_ref, m_ref, mx_ref)


@functools.lru_cache(maxsize=None)
def _make_mid(Dp, Dn, sum_parts):
    Dhp = Dp // 4 if not sum_parts else Dp
    body = functools.partial(_mid_body, Dp, Dn, sum_parts)
    return pl.pallas_call(
        body,
        grid=(GRID,),
        in_specs=[
            pl.BlockSpec((1, BLK, Dhp), lambda i: (0, i, 0)),
            pl.BlockSpec((1, BLK, Dhp), lambda i: (1, i, 0)),
            pl.BlockSpec((1, BLK, Dhp), lambda i: (0, i, 0)),
            pl.BlockSpec((1, BLK, Dhp), lambda i: (1, i, 0)),
            pl.BlockSpec((1, BLK, 1), lambda i: (0, i, 0)),
            pl.BlockSpec((1, BLK, 1), lambda i: (1, i, 0)),
            pl.BlockSpec((1, Dp), lambda i: (0, 0)),
            pl.BlockSpec((Dp, Dn), lambda i: (0, 0)),
            pl.BlockSpec((1, Dn), lambda i: (0, 0)),
            pl.BlockSpec((1, Dn), lambda i: (0, 0)),
        ],
        out_specs=[
            pl.BlockSpec((BLK, 16), lambda i: (i, 0)),
            pl.BlockSpec((BLK, 1), lambda i: (i, 0)),
            pl.BlockSpec((BLK, 1), lambda i: (i, 0)),
            pl.BlockSpec((1, 16), lambda i: (0, 0)),
        ],
        out_shape=[
            jax.ShapeDtypeStruct((N_PAD, 16), jnp.float32),
            jax.ShapeDtypeStruct((N_PAD, 1), jnp.float32),
            jax.ShapeDtypeStruct((N_PAD, 1), jnp.float32),
            jax.ShapeDtypeStruct((1, 16), jnp.float32),
        ],
        scratch_shapes=[pltpu.SMEM((2,), jnp.float32)],
    )


def _final_body(n0_ref, n1_ref, d0_ref, d1_ref, b_ref, o_ref):
    num = n0_ref[0] + n1_ref[0]
    den = d0_ref[0] + d1_ref[0]
    o = num[:, :2] / den + b_ref[...]
    o_ref[...] = jax.nn.sigmoid(o)


@jax.jit
def _tc_final(num3, den3, b3r):
    return pl.pallas_call(
        _final_body,
        grid=(GRID,),
        in_specs=[
            pl.BlockSpec((1, BLK, 16), lambda i: (0, i, 0)),
            pl.BlockSpec((1, BLK, 16), lambda i: (1, i, 0)),
            pl.BlockSpec((1, BLK, 1), lambda i: (0, i, 0)),
            pl.BlockSpec((1, BLK, 1), lambda i: (1, i, 0)),
            pl.BlockSpec((1, 2), lambda i: (0, 0)),
        ],
        out_specs=pl.BlockSpec((BLK, 2), lambda i: (i, 0)),
        out_shape=jax.ShapeDtypeStruct((N_PAD, 2), jnp.float32),
    )(num3, num3, den3, den3, b3r)


def kernel(x, edge_index, W1, a_src1, a_dst1, b1,
           W2, a_src2, a_dst2, b2, W3, a_src3, a_dst3, b3):
    ei = edge_index.astype(jnp.int32)
    loop = jnp.arange(N, dtype=jnp.int32)
    padz = jnp.zeros((E_PAD + S_BIG - E_TOT,), jnp.int32)
    srcp = jnp.concatenate([ei[0], loop, padz])
    dstp = jnp.concatenate([ei[1], loop, padz])

    x_p = jnp.pad(x, ((0, N_PAD - N), (0, 0)))

    h1s, as1, ad1, m1 = _tc1(x_p, W1, a_src1.reshape(1, 256), a_dst1.reshape(1, 256))
    h1t = h1s.reshape(4 * N_PAD, 64)
    as1r, ad1r, m1r = as1.reshape(-1), ad1.reshape(-1), m1.reshape(-1)
    num1a, den1 = _sc_agg(h1t, srcp, dstp, as1r, ad1r, m1r,
                          Dh=64, S=S_BIG, split_edges=False,
                          off0=0, offc=N_PAD, do_den=True)
    num1b, _ = _sc_agg(h1t, srcp, dstp, as1r, ad1r, m1r,
                       Dh=64, S=S_BIG, split_edges=False,
                       off0=2 * N_PAD, offc=N_PAD, do_den=False)

    h2, as2, ad2, m2 = _make_mid(256, 16, False)(
        num1a, num1a, num1b, num1b,
        den1.reshape(2, N_PAD, 1), den1.reshape(2, N_PAD, 1),
        b1.reshape(1, 256), W2, a_src2.reshape(1, 16), a_dst2.reshape(1, 16))
    num2, den2 = _sc_agg(h2, srcp, dstp,
                         as2.reshape(-1), ad2.reshape(-1), m2.reshape(-1),
                         Dh=16, S=S_BIG, split_edges=True, off0=0, offc=0)

    h3, as3, ad3, m3 = _make_mid(16, 2, True)(
        num2, num2, num2, num2,
        den2.reshape(2, N_PAD, 1), den2.reshape(2, N_PAD, 1),
        b2.reshape(1, 16), W3, a_src3.reshape(1, 2), a_dst3.reshape(1, 2))
    num3, den3 = _sc_agg(h3, srcp, dstp,
                         as3.reshape(-1), ad3.reshape(-1), m3.reshape(-1),
                         Dh=16, S=S_BIG, split_edges=True, off0=0, offc=0)

    out = _tc_final(num3, den3.reshape(2, N_PAD, 1), b3.reshape(1, 2))
    return out[:N]


# speedup vs baseline: 1.1791x; 1.1791x over previous
import functools

import jax
import jax.numpy as jnp
from jax import lax
from jax.experimental import pallas as pl
from jax.experimental.pallas import tpu as pltpu
from jax.experimental.pallas import tpu_sc as plsc

N = 10000
N_PAD = 10240
E_RAW = 320000
E_TOT = E_RAW + N
E_PAD = 331776
S_BIG = 384
BLK = 1024
GRID = N_PAD // BLK
STRIPE = N_PAD // 16


def _sc_agg_kernel(Dh, S, split_edges, off0, offc, do_den, unroll_scale, n_chunks,
                   h_hbm, src_hbm, dst_hbm, as_hbm, ad_hbm, m_hbm,
                   zn_hbm, zd_hbm,
                   num_hbm, den_hbm,
                   acc, dacc, srcv, srcav, dstv, rowsv, wv, asg, adg, mv,
                   g0, g1, g2, a0, a1, a2, b0, b1, b2,
                   s0, s1, s2, d0, d1, d2):
    c = lax.axis_index("c")
    s = lax.axis_index("s")
    gsem = [g0, g1, g2]
    asem = [a0, a1, a2]
    bsem = [b0, b1, b2]
    ssem = [s0, s1, s2]
    dsem = [d0, d1, d2]
    subs = [(o, min(128, S - o)) for o in range(0, S, 128)]
    adjust = (off0 != 0) or (offc != 0)

    pltpu.sync_copy(zn_hbm, acc.at[pl.ds(s * STRIPE, STRIPE)])
    if do_den:
        pltpu.sync_copy(zd_hbm, dacc.at[pl.ds(s * STRIPE, STRIPE)])
    pltpu.sync_copy(m_hbm, mv)

    plsc.subcore_barrier()

    if split_edges:
        per_tile = E_PAD // 32
        base0 = (c * 16 + s) * per_tile
    else:
        per_tile = E_PAD // 16
        base0 = s * per_tile

    def gidx(b, d):
        ref = srcav if adjust else srcv
        o, l = subs[d]
        return ref.at[b, pl.ds(o, l)]

    def stage_a(i, b):
        base = base0 + i * S
        pltpu.sync_copy(src_hbm.at[pl.ds(base, S)], srcv.at[b])
        for d, (o, l) in enumerate(subs):
            pltpu.sync_copy(dst_hbm.at[pl.ds(base + o, l)], dstv.at[b, d])
        if adjust:
            for j in range(S // 16):
                sl = pl.ds(j * 16, 16)
                srcav[b, sl] = srcv[b, sl] + (off0 + c * offc)
        for d, (o, l) in enumerate(subs):
            pltpu.async_copy(h_hbm.at[gidx(b, d)], rowsv.at[b, pl.ds(o, l)], gsem[b])
            pltpu.async_copy(as_hbm.at[srcv.at[b, pl.ds(o, l)]], asg.at[b, pl.ds(o, l)], asem[b])
            pltpu.async_copy(ad_hbm.at[dstv.at[b, d]], adg.at[b, pl.ds(o, l)], bsem[b])

    def stage_b(i, b):
        base = base0 + i * S
        pltpu.make_async_copy(as_hbm.at[pl.ds(0, S)], asg.at[b], asem[b]).wait()
        pltpu.make_async_copy(ad_hbm.at[pl.ds(0, S)], adg.at[b], bsem[b]).wait()
        m16 = mv[...]
        for j in range(S // 16):
            sl = pl.ds(j * 16, 16)
            e = asg[b, sl] + adg[b, sl]
            e = jnp.where(e > 0, e, 0.2 * e)
            w = jnp.exp(e - m16)
            eid = base + j * 16 + lax.iota(jnp.int32, 16)
            wv[b, sl] = jnp.where(eid < E_TOT, w, 0.0)
        pltpu.make_async_copy(h_hbm.at[pl.ds(0, S)], rowsv.at[b], gsem[b]).wait()

        if unroll_scale:
            for j in range(S // 16):
                w16 = wv[b, pl.ds(j * 16, 16)]
                for t in range(16):
                    ws = w16[t]
                    r = j * 16 + t
                    for k in range(Dh // 16):
                        cl = pl.ds(k * 16, 16)
                        rowsv[b, r, cl] = rowsv[b, r, cl] * ws
        else:
            def scale_group(g, carry):
                w16 = wv[b, pl.ds(g * 16, 16)]
                for t in range(16):
                    ws = w16[t]
                    r = g * 16 + t
                    for k in range(Dh // 16):
                        cl = pl.ds(k * 16, 16)
                        rowsv[b, r, cl] = rowsv[b, r, cl] * ws
                return carry

            lax.fori_loop(0, S // 16, scale_group, 0)
        for d, (o, l) in enumerate(subs):
            pltpu.async_copy(rowsv.at[b, pl.ds(o, l)], acc.at[dstv.at[b, d]],
                             ssem[b], add=True)
            if do_den:
                pltpu.async_copy(wv.at[b, pl.ds(o, l)], dacc.at[dstv.at[b, d]],
                                 dsem[b], add=True)

    def wait_scatters(b):
        pltpu.make_async_copy(h_hbm.at[pl.ds(0, S)], rowsv.at[b], ssem[b]).wait()
        if do_den:
            pltpu.make_async_copy(as_hbm.at[pl.ds(0, S)], wv.at[b], dsem[b]).wait()

    stage_a(0, 0)

    def outer(k, carry):
        for b in range(3):
            i = k * 3 + b
            bn = (b + 1) % 3
            if b == 2:
                wait_scatters(bn)
            else:
                @pl.when(k > 0)
                def _():
                    wait_scatters(bn)
            stage_a(i + 1, bn)
            stage_b(i, b)
        return carry

    lax.fori_loop(0, n_chunks // 3, outer, 0)

    wait_scatters((n_chunks - 2) % 3)
    wait_scatters((n_chunks - 1) % 3)
    bx = n_chunks % 3
    pltpu.make_async_copy(h_hbm.at[pl.ds(0, S)], rowsv.at[bx], gsem[bx]).wait()
    pltpu.make_async_copy(as_hbm.at[pl.ds(0, S)], asg.at[bx], asem[bx]).wait()
    pltpu.make_async_copy(ad_hbm.at[pl.ds(0, S)], adg.at[bx], bsem[bx]).wait()

    plsc.subcore_barrier()

    rs = pl.ds(s * STRIPE, STRIPE)
    pltpu.sync_copy(acc.at[rs], num_hbm.at[c, rs])
    if do_den:
        pltpu.sync_copy(dacc.at[rs], den_hbm.at[c, rs])


@functools.lru_cache(maxsize=None)
def _make_sc_agg(Dh, S, split_edges, off0, offc, do_den):
    n_chunks = (E_PAD // 32 if split_edges else E_PAD // 16) // S
    nd = (S + 127) // 128
    mesh = plsc.VectorSubcoreMesh(core_axis_name="c", subcore_axis_name="s")
    return pl.kernel(
        functools.partial(_sc_agg_kernel, Dh, S, split_edges, off0, offc,
                          do_den, S * Dh <= 96 * 128, n_chunks),
        mesh=mesh,
        out_type=[
            jax.ShapeDtypeStruct((2, N_PAD, Dh), jnp.float32),
            jax.ShapeDtypeStruct((2, N_PAD), jnp.float32),
        ],
        scratch_types=[
            pltpu.VMEM_SHARED((N_PAD, Dh), jnp.float32),
            pltpu.VMEM_SHARED((N_PAD,), jnp.float32),
            pltpu.VMEM((3, S), jnp.int32),
            pltpu.VMEM((3, S), jnp.int32),
            pltpu.VMEM((3, nd, min(128, S)), jnp.int32),
            pltpu.VMEM((3, S, Dh), jnp.float32),
            pltpu.VMEM((3, S), jnp.float32),
            pltpu.VMEM((3, S), jnp.float32),
            pltpu.VMEM((3, S), jnp.float32),
            pltpu.VMEM((16,), jnp.float32),
        ] + [pltpu.SemaphoreType.DMA] * 15,
        compiler_params=pltpu.CompilerParams(
            needs_layout_passes=False, use_tc_tiling_on_sc=False),
    )


def _sc_agg(h_table, srcp, dstp, as_t, ad_t, m16, Dh, S, split_edges,
            off0, offc, do_den=True):
    zn = jnp.zeros((STRIPE, Dh), jnp.float32)
    zd = jnp.zeros((STRIPE,), jnp.float32)
    fn = _make_sc_agg(Dh, S, split_edges, off0, offc, do_den)
    return fn(h_table, srcp, dstp, as_t, ad_t, m16, zn, zd)


def _leaky(t):
    return jnp.where(t > 0, t, 0.2 * t)


def _alphas_and_max(h, asr, adr, i, as_ref, ad_ref, m_ref, mx_ref):
    a_s = jnp.sum(h * asr, axis=1)
    a_d = jnp.sum(h * adr, axis=1)
    as_ref[...] = a_s[:, None]
    ad_ref[...] = a_d[:, None]
    bs = jnp.max(a_s)
    bd = jnp.max(a_d)

    @pl.when(i == 0)
    def _():
        mx_ref[0] = bs
        mx_ref[1] = bd

    @pl.when(i > 0)
    def _():
        mx_ref[0] = jnp.maximum(mx_ref[0], bs)
        mx_ref[1] = jnp.maximum(mx_ref[1], bd)

    m = _leaky(mx_ref[0] + mx_ref[1])
    m_ref[...] = jnp.full((1, 16), m, jnp.float32)


def _tc1_body(x_ref, w_ref, asr_ref, adr_ref,
              h_ref, as_ref, ad_ref, m_ref, mx_ref):
    i = pl.program_id(0)
    h = jnp.dot(x_ref[...], w_ref[...], preferred_element_type=jnp.float32)
    h_ref[0] = h[:, :128]
    h_ref[1] = h[:, 128:]
    _alphas_and_max(h, asr_ref[...], adr_ref[...], i, as_ref, ad_ref, m_ref, mx_ref)


@jax.jit
def _tc1(x_p, W1, asr, adr):
    return pl.pallas_call(
        _tc1_body,
        grid=(GRID,),
        in_specs=[
            pl.BlockSpec((BLK, 128), lambda i: (i, 0)),
            pl.BlockSpec((128, 256), lambda i: (0, 0)),
            pl.BlockSpec((1, 256), lambda i: (0, 0)),
            pl.BlockSpec((1, 256), lambda i: (0, 0)),
        ],
        out_specs=[
            pl.BlockSpec((2, BLK, 128), lambda i: (0, i, 0)),
            pl.BlockSpec((BLK, 1), lambda i: (i, 0)),
            pl.BlockSpec((BLK, 1), lambda i: (i, 0)),
            pl.BlockSpec((1, 16), lambda i: (0, 0)),
        ],
        out_shape=[
            jax.ShapeDtypeStruct((2, N_PAD, 128), jnp.float32),
            jax.ShapeDtypeStruct((N_PAD, 1), jnp.float32),
            jax.ShapeDtypeStruct((N_PAD, 1), jnp.float32),
            jax.ShapeDtypeStruct((1, 16), jnp.float32),
        ],
        scratch_shapes=[pltpu.SMEM((2,), jnp.float32)],
    )(x_p, W1, asr, adr)


def _mid_body(Dp, Dn, sum_parts, n0_ref, n1_ref,
              d0_ref, d1_ref, b_ref, w_ref,
              asr_ref, adr_ref, h_ref, as_ref, ad_ref, m_ref, mx_ref):
    i = pl.program_id(0)
    if sum_parts:
        num = n0_ref[0] + n1_ref[0]
        den = d0_ref[0] + d1_ref[0]
    else:
        num = jnp.concatenate([n0_ref[0], n1_ref[0]], axis=1)
        den = d0_ref[0]
    x = num / den + b_ref[...]
    x = jnp.maximum(x, 0.0)
    row = i * BLK + lax.broadcasted_iota(jnp.int32, (BLK, 1), 0)
    x = jnp.where(row < N, x, 0.0)
    h = jnp.dot(x, w_ref[...], preferred_element_type=jnp.float32)
    if Dn < 16:
        h_ref[...] = jnp.concatenate(
            [h, jnp.zeros((BLK, 16 - Dn), jnp.float32)], axis=1)
    else:
        h_ref[...] = h
    _alphas_and_max(h, asr_ref[...], adr_ref[...], i, as_ref, ad_ref, m_ref, mx_ref)


@functools.lru_cache(maxsize=None)
def _make_mid(Dp, Dn, sum_parts):
    Dhp = Dp // 2 if not sum_parts else Dp
    body = functools.partial(_mid_body, Dp, Dn, sum_parts)
    return pl.pallas_call(
        body,
        grid=(GRID,),
        in_specs=[
            pl.BlockSpec((1, BLK, Dhp), lambda i: (0, i, 0)),
            pl.BlockSpec((1, BLK, Dhp), lambda i: (1, i, 0)),
            pl.BlockSpec((1, BLK, 1), lambda i: (0, i, 0)),
            pl.BlockSpec((1, BLK, 1), lambda i: (1, i, 0)),
            pl.BlockSpec((1, Dp), lambda i: (0, 0)),
            pl.BlockSpec((Dp, Dn), lambda i: (0, 0)),
            pl.BlockSpec((1, Dn), lambda i: (0, 0)),
            pl.BlockSpec((1, Dn), lambda i: (0, 0)),
        ],
        out_specs=[
            pl.BlockSpec((BLK, 16), lambda i: (i, 0)),
            pl.BlockSpec((BLK, 1), lambda i: (i, 0)),
            pl.BlockSpec((BLK, 1), lambda i: (i, 0)),
            pl.BlockSpec((1, 16), lambda i: (0, 0)),
        ],
        out_shape=[
            jax.ShapeDtypeStruct((N_PAD, 16), jnp.float32),
            jax.ShapeDtypeStruct((N_PAD, 1), jnp.float32),
            jax.ShapeDtypeStruct((N_PAD, 1), jnp.float32),
            jax.ShapeDtypeStruct((1, 16), jnp.float32),
        ],
        scratch_shapes=[pltpu.SMEM((2,), jnp.float32)],
    )


def _final_body(n0_ref, n1_ref, d0_ref, d1_ref, b_ref, o_ref):
    num = n0_ref[0] + n1_ref[0]
    den = d0_ref[0] + d1_ref[0]
    o = num[:, :2] / den + b_ref[...]
    o_ref[...] = jax.nn.sigmoid(o)


@jax.jit
def _tc_final(num3, den3, b3r):
    return pl.pallas_call(
        _final_body,
        grid=(GRID,),
        in_specs=[
            pl.BlockSpec((1, BLK, 16), lambda i: (0, i, 0)),
            pl.BlockSpec((1, BLK, 16), lambda i: (1, i, 0)),
            pl.BlockSpec((1, BLK, 1), lambda i: (0, i, 0)),
            pl.BlockSpec((1, BLK, 1), lambda i: (1, i, 0)),
            pl.BlockSpec((1, 2), lambda i: (0, 0)),
        ],
        out_specs=pl.BlockSpec((BLK, 2), lambda i: (i, 0)),
        out_shape=jax.ShapeDtypeStruct((N_PAD, 2), jnp.float32),
    )(num3, num3, den3, den3, b3r)


def kernel(x, edge_index, W1, a_src1, a_dst1, b1,
           W2, a_src2, a_dst2, b2, W3, a_src3, a_dst3, b3):
    ei = edge_index.astype(jnp.int32)
    loop = jnp.arange(N, dtype=jnp.int32)
    padz = jnp.zeros((E_PAD + S_BIG - E_TOT,), jnp.int32)
    srcp = jnp.concatenate([ei[0], loop, padz])
    dstp = jnp.concatenate([ei[1], loop, padz])

    x_p = jnp.pad(x, ((0, N_PAD - N), (0, 0)))

    h1s, as1, ad1, m1 = _tc1(x_p, W1, a_src1.reshape(1, 256), a_dst1.reshape(1, 256))
    h1t = h1s.reshape(2 * N_PAD, 128)
    num1, den1 = _sc_agg(h1t, srcp, dstp,
                         as1.reshape(-1), ad1.reshape(-1), m1.reshape(-1),
                         Dh=128, S=96, split_edges=False,
                         off0=0, offc=N_PAD, do_den=True)

    h2, as2, ad2, m2 = _make_mid(256, 16, False)(
        num1, num1,
        den1.reshape(2, N_PAD, 1), den1.reshape(2, N_PAD, 1),
        b1.reshape(1, 256), W2, a_src2.reshape(1, 16), a_dst2.reshape(1, 16))
    num2, den2 = _sc_agg(h2, srcp, dstp,
                         as2.reshape(-1), ad2.reshape(-1), m2.reshape(-1),
                         Dh=16, S=S_BIG, split_edges=True, off0=0, offc=0)

    h3, as3, ad3, m3 = _make_mid(16, 2, True)(
        num2, num2,
        den2.reshape(2, N_PAD, 1), den2.reshape(2, N_PAD, 1),
        b2.reshape(1, 16), W3, a_src3.reshape(1, 2), a_dst3.reshape(1, 2))
    num3, den3 = _sc_agg(h3, srcp, dstp,
                         as3.reshape(-1), ad3.reshape(-1), m3.reshape(-1),
                         Dh=16, S=S_BIG, split_edges=True, off0=0, offc=0)

    out = _tc_final(num3, den3.reshape(2, N_PAD, 1), b3.reshape(1, 2))
    return out[:N]


# speedup vs baseline: 1.1846x; 1.0047x over previous
import functools

import jax
import jax.numpy as jnp
from jax import lax
from jax.experimental import pallas as pl
from jax.experimental.pallas import tpu as pltpu
from jax.experimental.pallas import tpu_sc as plsc

N = 10000
N_PAD = 10240
E_RAW = 320000
E_TOT = E_RAW + N
E_PAD = 331776
S_BIG = 384
BLK = 2048
GRID = N_PAD // BLK
STRIPE = N_PAD // 16


def _sc_agg_kernel(Dh, S, split_edges, off0, offc, do_den, unroll_scale, n_chunks,
                   h_hbm, src_hbm, dst_hbm, as_hbm, ad_hbm, m_hbm,
                   zn_hbm, zd_hbm,
                   num_hbm, den_hbm,
                   acc, dacc, srcv, srcav, dstv, rowsv, wv, asg, adg, mv,
                   g0, g1, g2, a0, a1, a2, b0, b1, b2,
                   s0, s1, s2, d0, d1, d2):
    c = lax.axis_index("c")
    s = lax.axis_index("s")
    gsem = [g0, g1, g2]
    asem = [a0, a1, a2]
    bsem = [b0, b1, b2]
    ssem = [s0, s1, s2]
    dsem = [d0, d1, d2]
    subs = [(o, min(128, S - o)) for o in range(0, S, 128)]
    adjust = (off0 != 0) or (offc != 0)

    pltpu.sync_copy(zn_hbm, acc.at[pl.ds(s * STRIPE, STRIPE)])
    if do_den:
        pltpu.sync_copy(zd_hbm, dacc.at[pl.ds(s * STRIPE, STRIPE)])
    pltpu.sync_copy(m_hbm, mv)

    plsc.subcore_barrier()

    if split_edges:
        per_tile = E_PAD // 32
        base0 = (c * 16 + s) * per_tile
    else:
        per_tile = E_PAD // 16
        base0 = s * per_tile

    def gidx(b, d):
        ref = srcav if adjust else srcv
        o, l = subs[d]
        return ref.at[b, pl.ds(o, l)]

    def stage_a(i, b):
        base = base0 + i * S
        pltpu.sync_copy(src_hbm.at[pl.ds(base, S)], srcv.at[b])
        for d, (o, l) in enumerate(subs):
            pltpu.sync_copy(dst_hbm.at[pl.ds(base + o, l)], dstv.at[b, d])
        if adjust:
            for j in range(S // 16):
                sl = pl.ds(j * 16, 16)
                srcav[b, sl] = srcv[b, sl] + (off0 + c * offc)
        for d, (o, l) in enumerate(subs):
            pltpu.async_copy(h_hbm.at[gidx(b, d)], rowsv.at[b, pl.ds(o, l)], gsem[b])
            pltpu.async_copy(as_hbm.at[srcv.at[b, pl.ds(o, l)]], asg.at[b, pl.ds(o, l)], asem[b])
            pltpu.async_copy(ad_hbm.at[dstv.at[b, d]], adg.at[b, pl.ds(o, l)], bsem[b])

    def stage_b(i, b):
        base = base0 + i * S
        pltpu.make_async_copy(as_hbm.at[pl.ds(0, S)], asg.at[b], asem[b]).wait()
        pltpu.make_async_copy(ad_hbm.at[pl.ds(0, S)], adg.at[b], bsem[b]).wait()
        m16 = mv[...]
        for j in range(S // 16):
            sl = pl.ds(j * 16, 16)
            e = asg[b, sl] + adg[b, sl]
            e = jnp.where(e > 0, e, 0.2 * e)
            w = jnp.exp(e - m16)
            eid = base + j * 16 + lax.iota(jnp.int32, 16)
            wv[b, sl] = jnp.where(eid < E_TOT, w, 0.0)
        pltpu.make_async_copy(h_hbm.at[pl.ds(0, S)], rowsv.at[b], gsem[b]).wait()

        if unroll_scale:
            for j in range(S // 16):
                w16 = wv[b, pl.ds(j * 16, 16)]
                for t in range(16):
                    ws = w16[t]
                    r = j * 16 + t
                    for k in range(Dh // 16):
                        cl = pl.ds(k * 16, 16)
                        rowsv[b, r, cl] = rowsv[b, r, cl] * ws
        else:
            def scale_group(g, carry):
                w16 = wv[b, pl.ds(g * 16, 16)]
                for t in range(16):
                    ws = w16[t]
                    r = g * 16 + t
                    for k in range(Dh // 16):
                        cl = pl.ds(k * 16, 16)
                        rowsv[b, r, cl] = rowsv[b, r, cl] * ws
                return carry

            lax.fori_loop(0, S // 16, scale_group, 0)
        for d, (o, l) in enumerate(subs):
            pltpu.async_copy(rowsv.at[b, pl.ds(o, l)], acc.at[dstv.at[b, d]],
                             ssem[b], add=True)
            if do_den:
                pltpu.async_copy(wv.at[b, pl.ds(o, l)], dacc.at[dstv.at[b, d]],
                                 dsem[b], add=True)

    def wait_scatters(b):
        pltpu.make_async_copy(h_hbm.at[pl.ds(0, S)], rowsv.at[b], ssem[b]).wait()
        if do_den:
            pltpu.make_async_copy(as_hbm.at[pl.ds(0, S)], wv.at[b], dsem[b]).wait()

    stage_a(0, 0)

    def outer(k, carry):
        for b in range(3):
            i = k * 3 + b
            bn = (b + 1) % 3
            if b == 2:
                wait_scatters(bn)
            else:
                @pl.when(k > 0)
                def _():
                    wait_scatters(bn)
            stage_a(i + 1, bn)
            stage_b(i, b)
        return carry

    lax.fori_loop(0, n_chunks // 3, outer, 0)

    wait_scatters((n_chunks - 2) % 3)
    wait_scatters((n_chunks - 1) % 3)
    bx = n_chunks % 3
    pltpu.make_async_copy(h_hbm.at[pl.ds(0, S)], rowsv.at[bx], gsem[bx]).wait()
    pltpu.make_async_copy(as_hbm.at[pl.ds(0, S)], asg.at[bx], asem[bx]).wait()
    pltpu.make_async_copy(ad_hbm.at[pl.ds(0, S)], adg.at[bx], bsem[bx]).wait()

    plsc.subcore_barrier()

    rs = pl.ds(s * STRIPE, STRIPE)
    pltpu.sync_copy(acc.at[rs], num_hbm.at[c, rs])
    if do_den:
        pltpu.sync_copy(dacc.at[rs], den_hbm.at[c, rs])


@functools.lru_cache(maxsize=None)
def _make_sc_agg(Dh, S, split_edges, off0, offc, do_den):
    n_chunks = (E_PAD // 32 if split_edges else E_PAD // 16) // S
    nd = (S + 127) // 128
    mesh = plsc.VectorSubcoreMesh(core_axis_name="c", subcore_axis_name="s")
    return pl.kernel(
        functools.partial(_sc_agg_kernel, Dh, S, split_edges, off0, offc,
                          do_den, S * Dh <= 96 * 128, n_chunks),
        mesh=mesh,
        out_type=[
            jax.ShapeDtypeStruct((2, N_PAD, Dh), jnp.float32),
            jax.ShapeDtypeStruct((2, N_PAD), jnp.float32),
        ],
        scratch_types=[
            pltpu.VMEM_SHARED((N_PAD, Dh), jnp.float32),
            pltpu.VMEM_SHARED((N_PAD,), jnp.float32),
            pltpu.VMEM((3, S), jnp.int32),
            pltpu.VMEM((3, S), jnp.int32),
            pltpu.VMEM((3, nd, min(128, S)), jnp.int32),
            pltpu.VMEM((3, S, Dh), jnp.float32),
            pltpu.VMEM((3, S), jnp.float32),
            pltpu.VMEM((3, S), jnp.float32),
            pltpu.VMEM((3, S), jnp.float32),
            pltpu.VMEM((16,), jnp.float32),
        ] + [pltpu.SemaphoreType.DMA] * 15,
        compiler_params=pltpu.CompilerParams(
            needs_layout_passes=False, use_tc_tiling_on_sc=False),
    )


def _sc_agg(h_table, srcp, dstp, as_t, ad_t, m16, Dh, S, split_edges,
            off0, offc, do_den=True):
    zn = jnp.zeros((STRIPE, Dh), jnp.float32)
    zd = jnp.zeros((STRIPE,), jnp.float32)
    fn = _make_sc_agg(Dh, S, split_edges, off0, offc, do_den)
    return fn(h_table, srcp, dstp, as_t, ad_t, m16, zn, zd)


def _leaky(t):
    return jnp.where(t > 0, t, 0.2 * t)


def _alphas_and_max(h, asr, adr, i, as_ref, ad_ref, m_ref, mx_ref):
    a_s = jnp.sum(h * asr, axis=1)
    a_d = jnp.sum(h * adr, axis=1)
    as_ref[...] = a_s[:, None]
    ad_ref[...] = a_d[:, None]
    bs = jnp.max(a_s)
    bd = jnp.max(a_d)

    @pl.when(i == 0)
    def _():
        mx_ref[0] = bs
        mx_ref[1] = bd

    @pl.when(i > 0)
    def _():
        mx_ref[0] = jnp.maximum(mx_ref[0], bs)
        mx_ref[1] = jnp.maximum(mx_ref[1], bd)

    m = _leaky(mx_ref[0] + mx_ref[1])
    m_ref[...] = jnp.full((1, 16), m, jnp.float32)


def _tc1_body(x_ref, w_ref, asr_ref, adr_ref,
              h_ref, as_ref, ad_ref, m_ref, mx_ref):
    i = pl.program_id(0)
    h = jnp.dot(x_ref[...], w_ref[...], preferred_element_type=jnp.float32)
    h_ref[0] = h[:, :128]
    h_ref[1] = h[:, 128:]
    _alphas_and_max(h, asr_ref[...], adr_ref[...], i, as_ref, ad_ref, m_ref, mx_ref)


@jax.jit
def _tc1(x_p, W1, asr, adr):
    return pl.pallas_call(
        _tc1_body,
        grid=(GRID,),
        in_specs=[
            pl.BlockSpec((BLK, 128), lambda i: (i, 0)),
            pl.BlockSpec((128, 256), lambda i: (0, 0)),
            pl.BlockSpec((1, 256), lambda i: (0, 0)),
            pl.BlockSpec((1, 256), lambda i: (0, 0)),
        ],
        out_specs=[
            pl.BlockSpec((2, BLK, 128), lambda i: (0, i, 0)),
            pl.BlockSpec((BLK, 1), lambda i: (i, 0)),
            pl.BlockSpec((BLK, 1), lambda i: (i, 0)),
            pl.BlockSpec((1, 16), lambda i: (0, 0)),
        ],
        out_shape=[
            jax.ShapeDtypeStruct((2, N_PAD, 128), jnp.float32),
            jax.ShapeDtypeStruct((N_PAD, 1), jnp.float32),
            jax.ShapeDtypeStruct((N_PAD, 1), jnp.float32),
            jax.ShapeDtypeStruct((1, 16), jnp.float32),
        ],
        scratch_shapes=[pltpu.SMEM((2,), jnp.float32)],
    )(x_p, W1, asr, adr)


def _mid_body(Dp, Dn, sum_parts, n0_ref, n1_ref,
              d0_ref, d1_ref, b_ref, w_ref,
              asr_ref, adr_ref, h_ref, as_ref, ad_ref, m_ref, mx_ref):
    i = pl.program_id(0)
    if sum_parts:
        num = n0_ref[0] + n1_ref[0]
        den = d0_ref[0] + d1_ref[0]
    else:
        num = jnp.concatenate([n0_ref[0], n1_ref[0]], axis=1)
        den = d0_ref[0]
    x = num / den + b_ref[...]
    x = jnp.maximum(x, 0.0)
    row = i * BLK + lax.broadcasted_iota(jnp.int32, (BLK, 1), 0)
    x = jnp.where(row < N, x, 0.0)
    h = jnp.dot(x, w_ref[...], preferred_element_type=jnp.float32)
    if Dn < 16:
        h_ref[...] = jnp.concatenate(
            [h, jnp.zeros((BLK, 16 - Dn), jnp.float32)], axis=1)
    else:
        h_ref[...] = h
    _alphas_and_max(h, asr_ref[...], adr_ref[...], i, as_ref, ad_ref, m_ref, mx_ref)


@functools.lru_cache(maxsize=None)
def _make_mid(Dp, Dn, sum_parts):
    Dhp = Dp // 2 if not sum_parts else Dp
    body = functools.partial(_mid_body, Dp, Dn, sum_parts)
    return pl.pallas_call(
        body,
        grid=(GRID,),
        in_specs=[
            pl.BlockSpec((1, BLK, Dhp), lambda i: (0, i, 0)),
            pl.BlockSpec((1, BLK, Dhp), lambda i: (1, i, 0)),
            pl.BlockSpec((1, BLK, 1), lambda i: (0, i, 0)),
            pl.BlockSpec((1, BLK, 1), lambda i: (1, i, 0)),
            pl.BlockSpec((1, Dp), lambda i: (0, 0)),
            pl.BlockSpec((Dp, Dn), lambda i: (0, 0)),
            pl.BlockSpec((1, Dn), lambda i: (0, 0)),
            pl.BlockSpec((1, Dn), lambda i: (0, 0)),
        ],
        out_specs=[
            pl.BlockSpec((BLK, 16), lambda i: (i, 0)),
            pl.BlockSpec((BLK, 1), lambda i: (i, 0)),
            pl.BlockSpec((BLK, 1), lambda i: (i, 0)),
            pl.BlockSpec((1, 16), lambda i: (0, 0)),
        ],
        out_shape=[
            jax.ShapeDtypeStruct((N_PAD, 16), jnp.float32),
            jax.ShapeDtypeStruct((N_PAD, 1), jnp.float32),
            jax.ShapeDtypeStruct((N_PAD, 1), jnp.float32),
            jax.ShapeDtypeStruct((1, 16), jnp.float32),
        ],
        scratch_shapes=[pltpu.SMEM((2,), jnp.float32)],
    )


def _final_body(n0_ref, n1_ref, d0_ref, d1_ref, b_ref, o_ref):
    num = n0_ref[0] + n1_ref[0]
    den = d0_ref[0] + d1_ref[0]
    o = num[:, :2] / den + b_ref[...]
    o_ref[...] = jax.nn.sigmoid(o)


@jax.jit
def _tc_final(num3, den3, b3r):
    return pl.pallas_call(
        _final_body,
        grid=(GRID,),
        in_specs=[
            pl.BlockSpec((1, BLK, 16), lambda i: (0, i, 0)),
            pl.BlockSpec((1, BLK, 16), lambda i: (1, i, 0)),
            pl.BlockSpec((1, BLK, 1), lambda i: (0, i, 0)),
            pl.BlockSpec((1, BLK, 1), lambda i: (1, i, 0)),
            pl.BlockSpec((1, 2), lambda i: (0, 0)),
        ],
        out_specs=pl.BlockSpec((BLK, 2), lambda i: (i, 0)),
        out_shape=jax.ShapeDtypeStruct((N_PAD, 2), jnp.float32),
    )(num3, num3, den3, den3, b3r)


def kernel(x, edge_index, W1, a_src1, a_dst1, b1,
           W2, a_src2, a_dst2, b2, W3, a_src3, a_dst3, b3):
    ei = edge_index.astype(jnp.int32)
    loop = jnp.arange(N, dtype=jnp.int32)
    padz = jnp.zeros((E_PAD + S_BIG - E_TOT,), jnp.int32)
    srcp = jnp.concatenate([ei[0], loop, padz])
    dstp = jnp.concatenate([ei[1], loop, padz])

    x_p = jnp.pad(x, ((0, N_PAD - N), (0, 0)))

    h1s, as1, ad1, m1 = _tc1(x_p, W1, a_src1.reshape(1, 256), a_dst1.reshape(1, 256))
    h1t = h1s.reshape(2 * N_PAD, 128)
    num1, den1 = _sc_agg(h1t, srcp, dstp,
                         as1.reshape(-1), ad1.reshape(-1), m1.reshape(-1),
                         Dh=128, S=96, split_edges=False,
                         off0=0, offc=N_PAD, do_den=True)

    h2, as2, ad2, m2 = _make_mid(256, 16, False)(
        num1, num1,
        den1.reshape(2, N_PAD, 1), den1.reshape(2, N_PAD, 1),
        b1.reshape(1, 256), W2, a_src2.reshape(1, 16), a_dst2.reshape(1, 16))
    num2, den2 = _sc_agg(h2, srcp, dstp,
                         as2.reshape(-1), ad2.reshape(-1), m2.reshape(-1),
                         Dh=16, S=S_BIG, split_edges=True, off0=0, offc=0)

    h3, as3, ad3, m3 = _make_mid(16, 2, True)(
        num2, num2,
        den2.reshape(2, N_PAD, 1), den2.reshape(2, N_PAD, 1),
        b2.reshape(1, 16), W3, a_src3.reshape(1, 2), a_dst3.reshape(1, 2))
    num3, den3 = _sc_agg(h3, srcp, dstp,
                         as3.reshape(-1), ad3.reshape(-1), m3.reshape(-1),
                         Dh=16, S=S_BIG, split_edges=True, off0=0, offc=0)

    out = _tc_final(num3, den3.reshape(2, N_PAD, 1), b3.reshape(1, 2))
    return out[:N]


# speedup vs baseline: 1.3112x; 1.1069x over previous
import functools

import jax
import jax.numpy as jnp
from jax import lax
from jax.experimental import pallas as pl
from jax.experimental.pallas import tpu as pltpu
from jax.experimental.pallas import tpu_sc as plsc

N = 10000
N_PAD = 10240
E_RAW = 320000
E_TOT = E_RAW + N
E_PAD = 331776
S_BIG = 384
BLK = 2048
GRID = N_PAD // BLK
STRIPE = N_PAD // 16


def _sc_agg_kernel(Dh, S, split_edges, off0, offc, do_den, unroll_scale, n_chunks,
                   h_hbm, src_hbm, dst_hbm, as_hbm, ad_hbm, m_hbm,
                   zn_hbm, zd_hbm,
                   num_hbm, den_hbm,
                   acc, dacc, srcv, srcav, dstv, rowsv, wv, asg, adg, mv,
                   g0, g1, g2, a0, a1, a2, b0, b1, b2,
                   s0, s1, s2, d0, d1, d2):
    c = lax.axis_index("c")
    s = lax.axis_index("s")
    gsem = [g0, g1, g2]
    asem = [a0, a1, a2]
    bsem = [b0, b1, b2]
    ssem = [s0, s1, s2]
    dsem = [d0, d1, d2]
    subs = [(o, min(128, S - o)) for o in range(0, S, 128)]
    adjust = (off0 != 0) or (offc != 0)

    pltpu.sync_copy(zn_hbm, acc.at[pl.ds(s * STRIPE, STRIPE)])
    if do_den:
        pltpu.sync_copy(zd_hbm, dacc.at[pl.ds(s * STRIPE, STRIPE)])
    pltpu.sync_copy(m_hbm, mv)

    plsc.subcore_barrier()

    if split_edges:
        per_tile = E_PAD // 32
        base0 = (c * 16 + s) * per_tile
    else:
        per_tile = E_PAD // 16
        base0 = s * per_tile

    def gidx(b, d):
        ref = srcav if adjust else srcv
        o, l = subs[d]
        return ref.at[b, pl.ds(o, l)]

    def stage_a(i, b):
        base = base0 + i * S
        pltpu.sync_copy(src_hbm.at[pl.ds(base, S)], srcv.at[b])
        for d, (o, l) in enumerate(subs):
            pltpu.sync_copy(dst_hbm.at[pl.ds(base + o, l)], dstv.at[b, d])
        if adjust:
            for j in range(S // 16):
                sl = pl.ds(j * 16, 16)
                srcav[b, sl] = srcv[b, sl] + (off0 + c * offc)
        for d, (o, l) in enumerate(subs):
            pltpu.async_copy(h_hbm.at[gidx(b, d)], rowsv.at[b, pl.ds(o, l)], gsem[b])
            pltpu.async_copy(as_hbm.at[srcv.at[b, pl.ds(o, l)]], asg.at[b, pl.ds(o, l)], asem[b])
            pltpu.async_copy(ad_hbm.at[dstv.at[b, d]], adg.at[b, pl.ds(o, l)], bsem[b])

    def stage_b(i, b):
        base = base0 + i * S
        pltpu.make_async_copy(as_hbm.at[pl.ds(0, S)], asg.at[b], asem[b]).wait()
        pltpu.make_async_copy(ad_hbm.at[pl.ds(0, S)], adg.at[b], bsem[b]).wait()
        m16 = mv[...]
        for j in range(S // 16):
            sl = pl.ds(j * 16, 16)
            e = asg[b, sl] + adg[b, sl]
            e = jnp.where(e > 0, e, 0.2 * e)
            w = jnp.exp(e - m16)
            eid = base + j * 16 + lax.iota(jnp.int32, 16)
            wv[b, sl] = jnp.where(eid < E_TOT, w, 0.0)
        pltpu.make_async_copy(h_hbm.at[pl.ds(0, S)], rowsv.at[b], gsem[b]).wait()

        @plsc.parallel_loop(0, S // 16, 1, unroll=2 if unroll_scale else 4)
        def _(g):
            w16 = wv[b, pl.ds(g * 16, 16)]
            for t in range(16):
                ws = w16[t]
                r = g * 16 + t
                for k in range(Dh // 16):
                    cl = pl.ds(k * 16, 16)
                    rowsv[b, r, cl] = rowsv[b, r, cl] * ws
        for d, (o, l) in enumerate(subs):
            pltpu.async_copy(rowsv.at[b, pl.ds(o, l)], acc.at[dstv.at[b, d]],
                             ssem[b], add=True)
            if do_den:
                pltpu.async_copy(wv.at[b, pl.ds(o, l)], dacc.at[dstv.at[b, d]],
                                 dsem[b], add=True)

    def wait_scatters(b):
        pltpu.make_async_copy(h_hbm.at[pl.ds(0, S)], rowsv.at[b], ssem[b]).wait()
        if do_den:
            pltpu.make_async_copy(as_hbm.at[pl.ds(0, S)], wv.at[b], dsem[b]).wait()

    stage_a(0, 0)

    def outer(k, carry):
        for b in range(3):
            i = k * 3 + b
            bn = (b + 1) % 3
            if b == 2:
                wait_scatters(bn)
            else:
                @pl.when(k > 0)
                def _():
                    wait_scatters(bn)
            stage_a(i + 1, bn)
            stage_b(i, b)
        return carry

    lax.fori_loop(0, n_chunks // 3, outer, 0)

    wait_scatters((n_chunks - 2) % 3)
    wait_scatters((n_chunks - 1) % 3)
    bx = n_chunks % 3
    pltpu.make_async_copy(h_hbm.at[pl.ds(0, S)], rowsv.at[bx], gsem[bx]).wait()
    pltpu.make_async_copy(as_hbm.at[pl.ds(0, S)], asg.at[bx], asem[bx]).wait()
    pltpu.make_async_copy(ad_hbm.at[pl.ds(0, S)], adg.at[bx], bsem[bx]).wait()

    plsc.subcore_barrier()

    rs = pl.ds(s * STRIPE, STRIPE)
    pltpu.sync_copy(acc.at[rs], num_hbm.at[c, rs])
    if do_den:
        pltpu.sync_copy(dacc.at[rs], den_hbm.at[c, rs])


@functools.lru_cache(maxsize=None)
def _make_sc_agg(Dh, S, split_edges, off0, offc, do_den):
    n_chunks = (E_PAD // 32 if split_edges else E_PAD // 16) // S
    nd = (S + 127) // 128
    mesh = plsc.VectorSubcoreMesh(core_axis_name="c", subcore_axis_name="s")
    return pl.kernel(
        functools.partial(_sc_agg_kernel, Dh, S, split_edges, off0, offc,
                          do_den, S * Dh <= 96 * 128, n_chunks),
        mesh=mesh,
        out_type=[
            jax.ShapeDtypeStruct((2, N_PAD, Dh), jnp.float32),
            jax.ShapeDtypeStruct((2, N_PAD), jnp.float32),
        ],
        scratch_types=[
            pltpu.VMEM_SHARED((N_PAD, Dh), jnp.float32),
            pltpu.VMEM_SHARED((N_PAD,), jnp.float32),
            pltpu.VMEM((3, S), jnp.int32),
            pltpu.VMEM((3, S), jnp.int32),
            pltpu.VMEM((3, nd, min(128, S)), jnp.int32),
            pltpu.VMEM((3, S, Dh), jnp.float32),
            pltpu.VMEM((3, S), jnp.float32),
            pltpu.VMEM((3, S), jnp.float32),
            pltpu.VMEM((3, S), jnp.float32),
            pltpu.VMEM((16,), jnp.float32),
        ] + [pltpu.SemaphoreType.DMA] * 15,
        compiler_params=pltpu.CompilerParams(
            needs_layout_passes=False, use_tc_tiling_on_sc=False),
    )


def _sc_agg(h_table, srcp, dstp, as_t, ad_t, m16, Dh, S, split_edges,
            off0, offc, do_den=True):
    zn = jnp.zeros((STRIPE, Dh), jnp.float32)
    zd = jnp.zeros((STRIPE,), jnp.float32)
    fn = _make_sc_agg(Dh, S, split_edges, off0, offc, do_den)
    return fn(h_table, srcp, dstp, as_t, ad_t, m16, zn, zd)


def _leaky(t):
    return jnp.where(t > 0, t, 0.2 * t)


def _alphas_and_max(h, asr, adr, i, as_ref, ad_ref, m_ref, mx_ref):
    a_s = jnp.sum(h * asr, axis=1)
    a_d = jnp.sum(h * adr, axis=1)
    as_ref[...] = a_s[:, None]
    ad_ref[...] = a_d[:, None]
    bs = jnp.max(a_s)
    bd = jnp.max(a_d)

    @pl.when(i == 0)
    def _():
        mx_ref[0] = bs
        mx_ref[1] = bd

    @pl.when(i > 0)
    def _():
        mx_ref[0] = jnp.maximum(mx_ref[0], bs)
        mx_ref[1] = jnp.maximum(mx_ref[1], bd)

    m = _leaky(mx_ref[0] + mx_ref[1])
    m_ref[...] = jnp.full((1, 16), m, jnp.float32)


def _tc1_body(x_ref, w_ref, asr_ref, adr_ref,
              h_ref, as_ref, ad_ref, m_ref, mx_ref):
    i = pl.program_id(0)
    h = jnp.dot(x_ref[...], w_ref[...], preferred_element_type=jnp.float32)
    h_ref[0] = h[:, :128]
    h_ref[1] = h[:, 128:]
    _alphas_and_max(h, asr_ref[...], adr_ref[...], i, as_ref, ad_ref, m_ref, mx_ref)


@jax.jit
def _tc1(x_p, W1, asr, adr):
    return pl.pallas_call(
        _tc1_body,
        grid=(GRID,),
        in_specs=[
            pl.BlockSpec((BLK, 128), lambda i: (i, 0)),
            pl.BlockSpec((128, 256), lambda i: (0, 0)),
            pl.BlockSpec((1, 256), lambda i: (0, 0)),
            pl.BlockSpec((1, 256), lambda i: (0, 0)),
        ],
        out_specs=[
            pl.BlockSpec((2, BLK, 128), lambda i: (0, i, 0)),
            pl.BlockSpec((BLK, 1), lambda i: (i, 0)),
            pl.BlockSpec((BLK, 1), lambda i: (i, 0)),
            pl.BlockSpec((1, 16), lambda i: (0, 0)),
        ],
        out_shape=[
            jax.ShapeDtypeStruct((2, N_PAD, 128), jnp.float32),
            jax.ShapeDtypeStruct((N_PAD, 1), jnp.float32),
            jax.ShapeDtypeStruct((N_PAD, 1), jnp.float32),
            jax.ShapeDtypeStruct((1, 16), jnp.float32),
        ],
        scratch_shapes=[pltpu.SMEM((2,), jnp.float32)],
    )(x_p, W1, asr, adr)


def _mid_body(Dp, Dn, sum_parts, n0_ref, n1_ref,
              d0_ref, d1_ref, b_ref, w_ref,
              asr_ref, adr_ref, h_ref, as_ref, ad_ref, m_ref, mx_ref):
    i = pl.program_id(0)
    if sum_parts:
        num = n0_ref[0] + n1_ref[0]
        den = d0_ref[0] + d1_ref[0]
    else:
        num = jnp.concatenate([n0_ref[0], n1_ref[0]], axis=1)
        den = d0_ref[0]
    x = num / den + b_ref[...]
    x = jnp.maximum(x, 0.0)
    row = i * BLK + lax.broadcasted_iota(jnp.int32, (BLK, 1), 0)
    x = jnp.where(row < N, x, 0.0)
    h = jnp.dot(x, w_ref[...], preferred_element_type=jnp.float32)
    if Dn < 16:
        h_ref[...] = jnp.concatenate(
            [h, jnp.zeros((BLK, 16 - Dn), jnp.float32)], axis=1)
    else:
        h_ref[...] = h
    _alphas_and_max(h, asr_ref[...], adr_ref[...], i, as_ref, ad_ref, m_ref, mx_ref)


@functools.lru_cache(maxsize=None)
def _make_mid(Dp, Dn, sum_parts):
    Dhp = Dp // 2 if not sum_parts else Dp
    body = functools.partial(_mid_body, Dp, Dn, sum_parts)
    return pl.pallas_call(
        body,
        grid=(GRID,),
        in_specs=[
            pl.BlockSpec((1, BLK, Dhp), lambda i: (0, i, 0)),
            pl.BlockSpec((1, BLK, Dhp), lambda i: (1, i, 0)),
            pl.BlockSpec((1, BLK, 1), lambda i: (0, i, 0)),
            pl.BlockSpec((1, BLK, 1), lambda i: (1, i, 0)),
            pl.BlockSpec((1, Dp), lambda i: (0, 0)),
            pl.BlockSpec((Dp, Dn), lambda i: (0, 0)),
            pl.BlockSpec((1, Dn), lambda i: (0, 0)),
            pl.BlockSpec((1, Dn), lambda i: (0, 0)),
        ],
        out_specs=[
            pl.BlockSpec((BLK, 16), lambda i: (i, 0)),
            pl.BlockSpec((BLK, 1), lambda i: (i, 0)),
            pl.BlockSpec((BLK, 1), lambda i: (i, 0)),
            pl.BlockSpec((1, 16), lambda i: (0, 0)),
        ],
        out_shape=[
            jax.ShapeDtypeStruct((N_PAD, 16), jnp.float32),
            jax.ShapeDtypeStruct((N_PAD, 1), jnp.float32),
            jax.ShapeDtypeStruct((N_PAD, 1), jnp.float32),
            jax.ShapeDtypeStruct((1, 16), jnp.float32),
        ],
        scratch_shapes=[pltpu.SMEM((2,), jnp.float32)],
    )


def _final_body(n0_ref, n1_ref, d0_ref, d1_ref, b_ref, o_ref):
    num = n0_ref[0] + n1_ref[0]
    den = d0_ref[0] + d1_ref[0]
    o = num[:, :2] / den + b_ref[...]
    o_ref[...] = jax.nn.sigmoid(o)


@jax.jit
def _tc_final(num3, den3, b3r):
    return pl.pallas_call(
        _final_body,
        grid=(GRID,),
        in_specs=[
            pl.BlockSpec((1, BLK, 16), lambda i: (0, i, 0)),
            pl.BlockSpec((1, BLK, 16), lambda i: (1, i, 0)),
            pl.BlockSpec((1, BLK, 1), lambda i: (0, i, 0)),
            pl.BlockSpec((1, BLK, 1), lambda i: (1, i, 0)),
            pl.BlockSpec((1, 2), lambda i: (0, 0)),
        ],
        out_specs=pl.BlockSpec((BLK, 2), lambda i: (i, 0)),
        out_shape=jax.ShapeDtypeStruct((N_PAD, 2), jnp.float32),
    )(num3, num3, den3, den3, b3r)


def kernel(x, edge_index, W1, a_src1, a_dst1, b1,
           W2, a_src2, a_dst2, b2, W3, a_src3, a_dst3, b3):
    ei = edge_index.astype(jnp.int32)
    loop = jnp.arange(N, dtype=jnp.int32)
    padz = jnp.zeros((E_PAD + S_BIG - E_TOT,), jnp.int32)
    srcp = jnp.concatenate([ei[0], loop, padz])
    dstp = jnp.concatenate([ei[1], loop, padz])

    x_p = jnp.pad(x, ((0, N_PAD - N), (0, 0)))

    h1s, as1, ad1, m1 = _tc1(x_p, W1, a_src1.reshape(1, 256), a_dst1.reshape(1, 256))
    h1t = h1s.reshape(2 * N_PAD, 128)
    num1, den1 = _sc_agg(h1t, srcp, dstp,
                         as1.reshape(-1), ad1.reshape(-1), m1.reshape(-1),
                         Dh=128, S=96, split_edges=False,
                         off0=0, offc=N_PAD, do_den=True)

    h2, as2, ad2, m2 = _make_mid(256, 16, False)(
        num1, num1,
        den1.reshape(2, N_PAD, 1), den1.reshape(2, N_PAD, 1),
        b1.reshape(1, 256), W2, a_src2.reshape(1, 16), a_dst2.reshape(1, 16))
    num2, den2 = _sc_agg(h2, srcp, dstp,
                         as2.reshape(-1), ad2.reshape(-1), m2.reshape(-1),
                         Dh=16, S=S_BIG, split_edges=True, off0=0, offc=0)

    h3, as3, ad3, m3 = _make_mid(16, 2, True)(
        num2, num2,
        den2.reshape(2, N_PAD, 1), den2.reshape(2, N_PAD, 1),
        b2.reshape(1, 16), W3, a_src3.reshape(1, 2), a_dst3.reshape(1, 2))
    num3, den3 = _sc_agg(h3, srcp, dstp,
                         as3.reshape(-1), ad3.reshape(-1), m3.reshape(-1),
                         Dh=16, S=S_BIG, split_edges=True, off0=0, offc=0)

    out = _tc_final(num3, den3.reshape(2, N_PAD, 1), b3.reshape(1, 2))
    return out[:N]
